# baseline (speedup 1.0000x reference)
import functools

import jax
import jax.numpy as jnp
from jax import lax
from jax.experimental import pallas as pl
from jax.experimental.pallas import tpu as pltpu
from jax.experimental.pallas import tpu_sc as plsc

N = 5000
E = 10000
B = 128
D = 64

NPAD = 5120
EPAD = 10240
EHALF = EPAD // 2
NCORES = 2
NSUB = 16
NW = NCORES * NSUB
EW = EHALF // NW
NR = NPAD // NSUB
CH = 32
NCH = EW // CH

ET = 256
EG = EPAD // ET
EGH = EHALF // ET
NT = 512
NG = NPAD // NT
ST = 640
SG = NPAD // ST


def _leaky(v):
    return jnp.where(v >= 0, v, 0.01 * v)


def _sc_gather_body(tbl_hbm, src_hbm, xj_hbm, idx_v, rows_v, sem):
    c = lax.axis_index("c")
    s = lax.axis_index("s")
    w = c * NSUB + s
    pltpu.sync_copy(src_hbm.at[w], idx_v)
    handles = [pltpu.async_copy(tbl_hbm.at[idx_v.at[j]],
                                rows_v.at[pl.ds(j * CH, CH)], sem)
               for j in range(NCH)]
    for h in handles:
        h.wait()
    pltpu.sync_copy(rows_v, xj_hbm.at[pl.ds(w * EW, EW)])


def _sc_scatter_body(msg_hbm, dst_hbm, zero_hbm, out_hbm,
                     idx_v, rows_v, tbl_sh, zsem, asem):
    c = lax.axis_index("c")
    s = lax.axis_index("s")
    w = c * NSUB + s
    zh = pltpu.async_copy(zero_hbm, tbl_sh.at[pl.ds(s * NR, NR)], zsem)
    pltpu.sync_copy(dst_hbm.at[w], idx_v)
    pltpu.sync_copy(msg_hbm.at[pl.ds(w * EW, EW)], rows_v)
    zh.wait()
    plsc.subcore_barrier()
    handles = [pltpu.async_copy(rows_v.at[pl.ds(j * CH, CH)],
                                tbl_sh.at[idx_v.at[j]], asem, add=True)
               for j in range(NCH)]
    for h in handles:
        h.wait()
    plsc.subcore_barrier()
    pltpu.sync_copy(tbl_sh.at[pl.ds(s * NR, NR)],
                    out_hbm.at[pl.ds(c * NPAD + s * NR, NR)])


@functools.lru_cache(maxsize=None)
def _sc_kernels():
    mesh = plsc.VectorSubcoreMesh(core_axis_name="c", subcore_axis_name="s",
                                  num_cores=NCORES, num_subcores=NSUB)
    params = pltpu.CompilerParams(use_tc_tiling_on_sc=False)
    gather = pl.kernel(
        _sc_gather_body,
        out_type=jax.ShapeDtypeStruct((EHALF, D), jnp.float32),
        mesh=mesh,
        compiler_params=params,
        scratch_types=[
            pltpu.VMEM((NCH, CH), jnp.int32),
            pltpu.VMEM((EW, D), jnp.float32),
            pltpu.SemaphoreType.DMA,
        ],
    )
    scatter = pl.kernel(
        _sc_scatter_body,
        out_type=jax.ShapeDtypeStruct((NCORES * NPAD, D), jnp.float32),
        mesh=mesh,
        compiler_params=params,
        scratch_types=[
            pltpu.VMEM((NCH, CH), jnp.int32),
            pltpu.VMEM((EW, D), jnp.float32),
            pltpu.VMEM_SHARED((NPAD, D), jnp.float32),
            pltpu.SemaphoreType.DMA,
            pltpu.SemaphoreType.DMA,
        ],
    )
    return gather, scatter


def _nodes_prologue_body(x_ref, w_ref, b_ref, cnt_ref, out_ref, cinv_ref):
    o = _leaky(jnp.dot(x_ref[...], w_ref[...],
                       preferred_element_type=jnp.float32) + b_ref[0:1])
    out_ref[...] = o
    c = (cnt_ref[0, :, 0:1] + cnt_ref[1, :, 0:1]
         + cnt_ref[2, :, 0:1] + cnt_ref[3, :, 0:1])
    cinv_ref[...] = jnp.broadcast_to(1.0 / jnp.maximum(c, 1.0), (NT, 8))


def _eh_body(ea_ref, w0_ref, b0_ref, out_ref):
    eh = _leaky(jnp.dot(ea_ref[...], w0_ref[...],
                        preferred_element_type=jnp.float32) + b0_ref[0:1])
    out_ref[...] = eh.astype(jnp.bfloat16)


def _msg_body(eha_ref, xj_ref, rep_ref, w2a_ref, msg_ref, g_s, xb_s):
    g_s[...] = jnp.dot(eha_ref[...], w2a_ref[...],
                       preferred_element_type=jnp.float32)
    xb_s[...] = jnp.dot(xj_ref[...].astype(jnp.bfloat16), rep_ref[...],
                        preferred_element_type=jnp.float32)
    for rb in range(ET // 64):
        acc = jnp.zeros((64, 128), jnp.float32)
        for j in range(D // 2):
            acc += (xb_s[pl.ds(rb * 64, 64), pl.ds(j * 128, 128)]
                    * g_s[pl.ds(rb * 64, 64), pl.ds(j * 128, 128)])
        msg_ref[pl.ds(rb * 64, 64), :] = acc[:, :D] + acc[:, D:]


def _gru_body(aggr_ref, cinv_ref, st_ref, root_ref, cb_ref,
              wir_ref, wiz_ref, win_ref, whr_ref, whz_ref, whn_ref,
              bir_ref, biz_ref, bin_ref, bhr_ref, bhz_ref, bhn_ref,
              new_ref):
    a = ((aggr_ref[0] + aggr_ref[1] + aggr_ref[2] + aggr_ref[3])
         * cinv_ref[:, 0:1])
    o = st_ref[...]
    m = _leaky(a + jnp.dot(o, root_ref[...],
                           preferred_element_type=jnp.float32) + cb_ref[0:1])
    gr = (jnp.dot(m, wir_ref[...], preferred_element_type=jnp.float32)
          + bir_ref[0:1]
          + jnp.dot(o, whr_ref[...], preferred_element_type=jnp.float32)
          + bhr_ref[0:1])
    gz = (jnp.dot(m, wiz_ref[...], preferred_element_type=jnp.float32)
          + biz_ref[0:1]
          + jnp.dot(o, whz_ref[...], preferred_element_type=jnp.float32)
          + bhz_ref[0:1])
    r = jax.nn.sigmoid(gr)
    z = jax.nn.sigmoid(gz)
    gn = (jnp.dot(m, win_ref[...], preferred_element_type=jnp.float32)
          + bin_ref[0:1]
          + r * (jnp.dot(o, whn_ref[...], preferred_element_type=jnp.float32)
                 + bhn_ref[0:1]))
    n = jnp.tanh(gn)
    new_ref[...] = (1.0 - z) * n + z * o


def _set2set_body(st_ref, bcol_ref, brow_ref, bih_ref, bhh_ref,
                  q_ref, r_ref, emax_s, den_s, r_s):
    p = pl.program_id(0)
    i = pl.program_id(1)
    gb = bih_ref[...] + bhh_ref[...]
    cl = jax.nn.sigmoid(gb[0:1]) * jnp.tanh(gb[2:3])
    qrow = jax.nn.sigmoid(gb[3:4]) * jnp.tanh(cl)

    o = st_ref[...]
    bt = bcol_ref[...]
    oh = bt == lax.broadcasted_iota(jnp.int32, (ST, B), 1)
    e = jnp.sum(o * qrow, axis=1, keepdims=True)

    @pl.when(p == 0)
    def _pass_max():
        @pl.when(i == 0)
        def _():
            emax_s[...] = jnp.full((8, B), -1e30, jnp.float32)
        part = jnp.max(jnp.where(oh, e, -1e30), axis=0, keepdims=True)
        emax_s[...] = jnp.maximum(emax_s[...], jnp.broadcast_to(part, (8, B)))

    @pl.when(p == 1)
    def _pass_read():
        @pl.when(i == 0)
        def _():
            den_s[...] = jnp.zeros((B, 8), jnp.float32)
            r_s[...] = jnp.zeros((B, D), jnp.float32)
        em = emax_s[0:1]
        eb = jnp.sum(jnp.where(oh, jnp.broadcast_to(em, (ST, B)), 0.0),
                     axis=1, keepdims=True)
        a = jnp.where(bt < B, jnp.exp(e - eb), 0.0)
        btr = brow_ref[0]
        oht = (lax.broadcasted_iota(jnp.int32, (B, ST), 0) == btr
               ).astype(jnp.float32)
        den_s[:, 0:1] += jnp.dot(oht, a, preferred_element_type=jnp.float32)
        r_s[...] += jnp.dot(oht, a * o, preferred_element_type=jnp.float32)

        @pl.when(i == SG - 1)
        def _():
            d = jnp.maximum(den_s[:, 0:1], 1e-16)
            r_ref[...] = r_s[...] / d
            q_ref[...] = jnp.broadcast_to(qrow, (B, D))


def _full(shape):
    return pl.BlockSpec(shape, lambda *_: tuple(0 for _ in shape))


_nodes_prologue = pl.pallas_call(
    _nodes_prologue_body,
    grid=(NG,),
    in_specs=[
        pl.BlockSpec((NT, 128), lambda i: (i, 0)),
        _full((128, D)),
        _full((8, D)),
        pl.BlockSpec((4, NT, D), lambda i: (0, i, 0)),
    ],
    out_specs=[
        pl.BlockSpec((NT, D), lambda i: (i, 0)),
        pl.BlockSpec((NT, 8), lambda i: (i, 0)),
    ],
    out_shape=[
        jax.ShapeDtypeStruct((NPAD, D), jnp.float32),
        jax.ShapeDtypeStruct((NPAD, 8), jnp.float32),
    ],
)

_eh = pl.pallas_call(
    _eh_body,
    grid=(EG,),
    in_specs=[
        pl.BlockSpec((ET, 8), lambda i: (i, 0)),
        _full((8, D)),
        _full((8, D)),
    ],
    out_specs=pl.BlockSpec((ET, D), lambda i: (i, 0)),
    out_shape=jax.ShapeDtypeStruct((EPAD, D), jnp.bfloat16),
)

_msg = pl.pallas_call(
    _msg_body,
    grid=(EGH,),
    in_specs=[
        pl.BlockSpec((ET, 128), lambda i: (i, 0)),
        pl.BlockSpec((ET, D), lambda i: (i, 0)),
        _full((D, D * D)),
        _full((128, D * D)),
    ],
    out_specs=pl.BlockSpec((ET, D), lambda i: (i, 0)),
    out_shape=jax.ShapeDtypeStruct((EHALF, D), jnp.float32),
    scratch_shapes=[pltpu.VMEM((ET, D * D), jnp.float32),
                    pltpu.VMEM((ET, D * D), jnp.float32)],
)

_gru = pl.pallas_call(
    _gru_body,
    grid=(NG,),
    in_specs=[
        pl.BlockSpec((4, NT, D), lambda i: (0, i, 0)),
        pl.BlockSpec((NT, 8), lambda i: (i, 0)),
        pl.BlockSpec((NT, D), lambda i: (i, 0)),
        _full((D, D)), _full((8, D)),
        _full((D, D)), _full((D, D)), _full((D, D)),
        _full((D, D)), _full((D, D)), _full((D, D)),
        _full((8, D)), _full((8, D)), _full((8, D)),
        _full((8, D)), _full((8, D)), _full((8, D)),
    ],
    out_specs=pl.BlockSpec((NT, D), lambda i: (i, 0)),
    out_shape=jax.ShapeDtypeStruct((NPAD, D), jnp.float32),
)

_set2set = pl.pallas_call(
    _set2set_body,
    grid=(2, SG),
    in_specs=[
        pl.BlockSpec((ST, D), lambda p, i: (i, 0)),
        pl.BlockSpec((ST, 1), lambda p, i: (i, 0)),
        pl.BlockSpec((1, 1, ST), lambda p, i: (i, 0, 0)),
        _full((8, D)),
        _full((8, D)),
    ],
    out_specs=[
        pl.BlockSpec((B, D), lambda p, i: (0, 0)),
        pl.BlockSpec((B, D), lambda p, i: (0, 0)),
    ],
    out_shape=[
        jax.ShapeDtypeStruct((B, D), jnp.float32),
        jax.ShapeDtypeStruct((B, D), jnp.float32),
    ],
    scratch_shapes=[
        pltpu.VMEM((8, B), jnp.float32),
        pltpu.VMEM((B, 8), jnp.float32),
        pltpu.VMEM((B, D), jnp.float32),
    ],
)


def kernel(x, edge_index, edge_attr, batch, lin0_W, lin0_b, net0_W, net0_b,
           net2_W, net2_b, root_W, conv_b, gru_Wih, gru_Whh, gru_bih, gru_bhh,
           lstm_Wih, lstm_Whh, lstm_bih, lstm_bhh):
    f32 = jnp.float32

    he = E // 2
    hp = EHALF - he
    def _half_idx(v, fill):
        return jnp.pad(v, (0, hp), constant_values=fill).reshape(NW, NCH, CH)
    src1 = _half_idx(edge_index[0, :he], 0)
    src2 = _half_idx(edge_index[0, he:], 0)
    dst1 = _half_idx(edge_index[1, :he], NPAD - 1)
    dst2 = _half_idx(edge_index[1, he:], NPAD - 1)
    xp = jnp.pad(x, ((0, NPAD - N), (0, 128 - x.shape[1])))
    eap = jnp.concatenate(
        [jnp.pad(edge_attr[:he], ((0, hp), (0, 8 - edge_attr.shape[1]))),
         jnp.pad(edge_attr[he:], ((0, hp), (0, 8 - edge_attr.shape[1])))],
        axis=0)
    ones_e = jnp.ones((EHALF, D), f32)
    zero_rows = jnp.zeros((NR, D), f32)
    rep = jnp.repeat(jnp.eye(D, dtype=jnp.bfloat16), D, axis=1)

    l0wt = jnp.pad(lin0_W.T, ((0, 128 - lin0_W.shape[1]), (0, 0)))
    l0b = jnp.broadcast_to(lin0_b, (8, D))
    w0t = jnp.pad(net0_W.T, ((0, 8 - net0_W.shape[1]), (0, 0)))
    b0 = jnp.broadcast_to(net0_b, (8, D))
    w2a = jnp.pad(jnp.concatenate([net2_W.T, net2_b[None, :]], axis=0),
                  ((0, 63), (0, 0))).astype(jnp.bfloat16)
    cb = jnp.broadcast_to(conv_b, (8, D))
    wir, wiz, win = (gru_Wih[i * D:(i + 1) * D].T for i in range(3))
    whr, whz, whn = (gru_Whh[i * D:(i + 1) * D].T for i in range(3))
    bir, biz, bin_ = (jnp.broadcast_to(gru_bih[i * D:(i + 1) * D], (8, D))
                      for i in range(3))
    bhr, bhz, bhn = (jnp.broadcast_to(gru_bhh[i * D:(i + 1) * D], (8, D))
                     for i in range(3))
    bih4 = jnp.pad(lstm_bih.reshape(4, D), ((0, 4), (0, 0)))
    bhh4 = jnp.pad(lstm_bhh.reshape(4, D), ((0, 4), (0, 0)))
    bpad = jnp.pad(batch, (0, NPAD - N), constant_values=2 * B)
    bcol = bpad.reshape(NPAD, 1)
    brow = bpad.reshape(SG, 1, ST)

    sc_gather, sc_scatter = _sc_kernels()
    c1 = sc_scatter(ones_e, dst1, zero_rows).reshape(NCORES, NPAD, D)
    c2 = sc_scatter(ones_e, dst2, zero_rows).reshape(NCORES, NPAD, D)
    cnt4 = jnp.concatenate([c1, c2], axis=0)
    out, cinv = _nodes_prologue(xp, l0wt, l0b, cnt4)
    eh64 = _eh(eap, w0t, b0)
    eha = jnp.concatenate(
        [eh64, jnp.ones((EPAD, 1), jnp.bfloat16),
         jnp.zeros((EPAD, 63), jnp.bfloat16)], axis=1)
    eha1, eha2 = eha[:EHALF], eha[EHALF:]

    for _ in range(6):
        xj1 = sc_gather(out, src1)
        xj2 = sc_gather(out, src2)
        msg1 = _msg(eha1, xj1, rep, w2a)
        a1 = sc_scatter(msg1, dst1, zero_rows).reshape(NCORES, NPAD, D)
        msg2 = _msg(eha2, xj2, rep, w2a)
        a2 = sc_scatter(msg2, dst2, zero_rows).reshape(NCORES, NPAD, D)
        aggr4 = jnp.concatenate([a1, a2], axis=0)
        out = _gru(aggr4, cinv, out, root_W, cb,
                   wir, wiz, win, whr, whz, whn,
                   bir, biz, bin_, bhr, bhz, bhn)

    q, r = _set2set(out, bcol, brow, bih4, bhh4)
    return jnp.concatenate([q, r], axis=1)


# speedup vs baseline: 1.0707x; 1.0707x over previous
import functools

import jax
import jax.numpy as jnp
from jax import lax
from jax.experimental import pallas as pl
from jax.experimental.pallas import tpu as pltpu
from jax.experimental.pallas import tpu_sc as plsc

N = 5000
E = 10000
B = 128
D = 64

NPAD = 5120
EPAD = 10240
NCORES = 2
NSUB = 16
NW = NCORES * NSUB
EW = EPAD // NW
NR = NPAD // NSUB
CH = 64
NCH = EW // CH

ET = 256
EG = EPAD // ET
NT = 512
NG = NPAD // NT
ST = 640
SG = NPAD // ST


def _leaky(v):
    return jnp.where(v >= 0, v, 0.01 * v)


def _sc_gather_body(tbl_hbm, src_hbm, xj_hbm, idx_v, rows_v, sem):
    c = lax.axis_index("c")
    s = lax.axis_index("s")
    w = c * NSUB + s
    pltpu.sync_copy(src_hbm.at[w], idx_v)
    handles = [pltpu.async_copy(tbl_hbm.at[idx_v.at[j]],
                                rows_v.at[pl.ds(j * CH, CH)], sem)
               for j in range(NCH)]
    for h in handles:
        h.wait()
    pltpu.sync_copy(rows_v, xj_hbm.at[pl.ds(w * EW, EW)])


def _sc_scatter_body(msg_hbm, dst_hbm, zero_hbm, out_hbm,
                     idx_v, rows_v, tbl_sh, zsem, asem):
    c = lax.axis_index("c")
    s = lax.axis_index("s")
    w = c * NSUB + s
    zh = pltpu.async_copy(zero_hbm, tbl_sh.at[pl.ds(s * NR, NR)], zsem)
    pltpu.sync_copy(dst_hbm.at[w], idx_v)
    pltpu.sync_copy(msg_hbm.at[pl.ds(w * EW, EW)], rows_v)
    zh.wait()
    plsc.subcore_barrier()
    handles = [pltpu.async_copy(rows_v.at[pl.ds(j * CH, CH)],
                                tbl_sh.at[idx_v.at[j]], asem, add=True)
               for j in range(NCH)]
    for h in handles:
        h.wait()
    plsc.subcore_barrier()
    pltpu.sync_copy(tbl_sh.at[pl.ds(s * NR, NR)],
                    out_hbm.at[pl.ds(c * NPAD + s * NR, NR)])


@functools.lru_cache(maxsize=None)
def _sc_kernels():
    mesh = plsc.VectorSubcoreMesh(core_axis_name="c", subcore_axis_name="s",
                                  num_cores=NCORES, num_subcores=NSUB)
    params = pltpu.CompilerParams(use_tc_tiling_on_sc=False)
    gather = pl.kernel(
        _sc_gather_body,
        out_type=jax.ShapeDtypeStruct((EPAD, D), jnp.float32),
        mesh=mesh,
        compiler_params=params,
        scratch_types=[
            pltpu.VMEM((NCH, CH), jnp.int32),
            pltpu.VMEM((EW, D), jnp.float32),
            pltpu.SemaphoreType.DMA,
        ],
    )
    scatter = pl.kernel(
        _sc_scatter_body,
        out_type=jax.ShapeDtypeStruct((NCORES * NPAD, D), jnp.float32),
        mesh=mesh,
        compiler_params=params,
        scratch_types=[
            pltpu.VMEM((NCH, CH), jnp.int32),
            pltpu.VMEM((EW, D), jnp.float32),
            pltpu.VMEM_SHARED((NPAD, D), jnp.float32),
            pltpu.SemaphoreType.DMA,
            pltpu.SemaphoreType.DMA,
        ],
    )
    return gather, scatter


def _nodes_prologue_body(x_ref, w_ref, b_ref, cnt_ref, out_ref, cinv_ref):
    o = _leaky(jnp.dot(x_ref[...], w_ref[...],
                       preferred_element_type=jnp.float32) + b_ref[0:1])
    out_ref[...] = o
    c = cnt_ref[0, :, 0:1] + cnt_ref[1, :, 0:1]
    cinv_ref[...] = jnp.broadcast_to(1.0 / jnp.maximum(c, 1.0), (NT, 8))


def _eh_body(ea_ref, w0_ref, b0_ref, out_ref):
    eh = _leaky(jnp.dot(ea_ref[...], w0_ref[...],
                        preferred_element_type=jnp.float32) + b0_ref[0:1])
    out_ref[...] = eh.astype(jnp.bfloat16)


def _wmat_body(eha_ref, w2a_ref, out_ref):
    out_ref[...] = jnp.dot(eha_ref[...], w2a_ref[...],
                           preferred_element_type=jnp.float32
                           ).astype(jnp.bfloat16)


def _msg_body(w_ref, xj_ref, rep_ref, msg_ref, xb_s):
    xb_s[...] = jnp.dot(xj_ref[...].astype(jnp.bfloat16), rep_ref[...],
                        preferred_element_type=jnp.float32)
    for rb in range(ET // 64):
        acc = jnp.zeros((64, 128), jnp.float32)
        for j in range(D // 2):
            acc += (xb_s[pl.ds(rb * 64, 64), pl.ds(j * 128, 128)]
                    * w_ref[pl.ds(rb * 64, 64),
                            pl.ds(j * 128, 128)].astype(jnp.float32))
        msg_ref[pl.ds(rb * 64, 64), :] = acc[:, :D] + acc[:, D:]


def _gru_body(aggr_ref, cinv_ref, st_ref, root_ref, cb_ref,
              wir_ref, wiz_ref, win_ref, whr_ref, whz_ref, whn_ref,
              bir_ref, biz_ref, bin_ref, bhr_ref, bhz_ref, bhn_ref,
              new_ref):
    a = (aggr_ref[0] + aggr_ref[1]) * cinv_ref[:, 0:1]
    o = st_ref[...]
    m = _leaky(a + jnp.dot(o, root_ref[...],
                           preferred_element_type=jnp.float32) + cb_ref[0:1])
    gr = (jnp.dot(m, wir_ref[...], preferred_element_type=jnp.float32)
          + bir_ref[0:1]
          + jnp.dot(o, whr_ref[...], preferred_element_type=jnp.float32)
          + bhr_ref[0:1])
    gz = (jnp.dot(m, wiz_ref[...], preferred_element_type=jnp.float32)
          + biz_ref[0:1]
          + jnp.dot(o, whz_ref[...], preferred_element_type=jnp.float32)
          + bhz_ref[0:1])
    r = jax.nn.sigmoid(gr)
    z = jax.nn.sigmoid(gz)
    gn = (jnp.dot(m, win_ref[...], preferred_element_type=jnp.float32)
          + bin_ref[0:1]
          + r * (jnp.dot(o, whn_ref[...], preferred_element_type=jnp.float32)
                 + bhn_ref[0:1]))
    n = jnp.tanh(gn)
    new_ref[...] = (1.0 - z) * n + z * o


def _set2set_body(st_ref, bcol_ref, brow_ref, bih_ref, bhh_ref,
                  q_ref, r_ref, emax_s, den_s, r_s):
    p = pl.program_id(0)
    i = pl.program_id(1)
    gb = bih_ref[...] + bhh_ref[...]
    cl = jax.nn.sigmoid(gb[0:1]) * jnp.tanh(gb[2:3])
    qrow = jax.nn.sigmoid(gb[3:4]) * jnp.tanh(cl)

    o = st_ref[...]
    bt = bcol_ref[...]
    oh = bt == lax.broadcasted_iota(jnp.int32, (ST, B), 1)
    e = jnp.sum(o * qrow, axis=1, keepdims=True)

    @pl.when(p == 0)
    def _pass_max():
        @pl.when(i == 0)
        def _():
            emax_s[...] = jnp.full((8, B), -1e30, jnp.float32)
        part = jnp.max(jnp.where(oh, e, -1e30), axis=0, keepdims=True)
        emax_s[...] = jnp.maximum(emax_s[...], jnp.broadcast_to(part, (8, B)))

    @pl.when(p == 1)
    def _pass_read():
        @pl.when(i == 0)
        def _():
            den_s[...] = jnp.zeros((B, 8), jnp.float32)
            r_s[...] = jnp.zeros((B, D), jnp.float32)
        em = emax_s[0:1]
        eb = jnp.sum(jnp.where(oh, jnp.broadcast_to(em, (ST, B)), 0.0),
                     axis=1, keepdims=True)
        a = jnp.where(bt < B, jnp.exp(e - eb), 0.0)
        btr = brow_ref[0]
        oht = (lax.broadcasted_iota(jnp.int32, (B, ST), 0) == btr
               ).astype(jnp.float32)
        den_s[:, 0:1] += jnp.dot(oht, a, preferred_element_type=jnp.float32)
        r_s[...] += jnp.dot(oht, a * o, preferred_element_type=jnp.float32)

        @pl.when(i == SG - 1)
        def _():
            d = jnp.maximum(den_s[:, 0:1], 1e-16)
            r_ref[...] = r_s[...] / d
            q_ref[...] = jnp.broadcast_to(qrow, (B, D))


def _full(shape):
    return pl.BlockSpec(shape, lambda *_: tuple(0 for _ in shape))


_nodes_prologue = pl.pallas_call(
    _nodes_prologue_body,
    grid=(NG,),
    in_specs=[
        pl.BlockSpec((NT, 128), lambda i: (i, 0)),
        _full((128, D)),
        _full((8, D)),
        pl.BlockSpec((2, NT, D), lambda i: (0, i, 0)),
    ],
    out_specs=[
        pl.BlockSpec((NT, D), lambda i: (i, 0)),
        pl.BlockSpec((NT, 8), lambda i: (i, 0)),
    ],
    out_shape=[
        jax.ShapeDtypeStruct((NPAD, D), jnp.float32),
        jax.ShapeDtypeStruct((NPAD, 8), jnp.float32),
    ],
)

_eh = pl.pallas_call(
    _eh_body,
    grid=(EG,),
    in_specs=[
        pl.BlockSpec((ET, 8), lambda i: (i, 0)),
        _full((8, D)),
        _full((8, D)),
    ],
    out_specs=pl.BlockSpec((ET, D), lambda i: (i, 0)),
    out_shape=jax.ShapeDtypeStruct((EPAD, D), jnp.bfloat16),
)

_wmat = pl.pallas_call(
    _wmat_body,
    grid=(EG,),
    in_specs=[
        pl.BlockSpec((ET, 128), lambda i: (i, 0)),
        _full((128, D * D)),
    ],
    out_specs=pl.BlockSpec((ET, D * D), lambda i: (i, 0)),
    out_shape=jax.ShapeDtypeStruct((EPAD, D * D), jnp.bfloat16),
)

_msg = pl.pallas_call(
    _msg_body,
    grid=(EG,),
    in_specs=[
        pl.BlockSpec((ET, D * D), lambda i: (i, 0)),
        pl.BlockSpec((ET, D), lambda i: (i, 0)),
        _full((D, D * D)),
    ],
    out_specs=pl.BlockSpec((ET, D), lambda i: (i, 0)),
    out_shape=jax.ShapeDtypeStruct((EPAD, D), jnp.float32),
    scratch_shapes=[pltpu.VMEM((ET, D * D), jnp.float32)],
)

_gru = pl.pallas_call(
    _gru_body,
    grid=(NG,),
    in_specs=[
        pl.BlockSpec((2, NT, D), lambda i: (0, i, 0)),
        pl.BlockSpec((NT, 8), lambda i: (i, 0)),
        pl.BlockSpec((NT, D), lambda i: (i, 0)),
        _full((D, D)), _full((8, D)),
        _full((D, D)), _full((D, D)), _full((D, D)),
        _full((D, D)), _full((D, D)), _full((D, D)),
        _full((8, D)), _full((8, D)), _full((8, D)),
        _full((8, D)), _full((8, D)), _full((8, D)),
    ],
    out_specs=pl.BlockSpec((NT, D), lambda i: (i, 0)),
    out_shape=jax.ShapeDtypeStruct((NPAD, D), jnp.float32),
)

_set2set = pl.pallas_call(
    _set2set_body,
    grid=(2, SG),
    in_specs=[
        pl.BlockSpec((ST, D), lambda p, i: (i, 0)),
        pl.BlockSpec((ST, 1), lambda p, i: (i, 0)),
        pl.BlockSpec((1, 1, ST), lambda p, i: (i, 0, 0)),
        _full((8, D)),
        _full((8, D)),
    ],
    out_specs=[
        pl.BlockSpec((B, D), lambda p, i: (0, 0)),
        pl.BlockSpec((B, D), lambda p, i: (0, 0)),
    ],
    out_shape=[
        jax.ShapeDtypeStruct((B, D), jnp.float32),
        jax.ShapeDtypeStruct((B, D), jnp.float32),
    ],
    scratch_shapes=[
        pltpu.VMEM((8, B), jnp.float32),
        pltpu.VMEM((B, 8), jnp.float32),
        pltpu.VMEM((B, D), jnp.float32),
    ],
)


def kernel(x, edge_index, edge_attr, batch, lin0_W, lin0_b, net0_W, net0_b,
           net2_W, net2_b, root_W, conv_b, gru_Wih, gru_Whh, gru_bih, gru_bhh,
           lstm_Wih, lstm_Whh, lstm_bih, lstm_bhh):
    f32 = jnp.float32

    src = jnp.pad(edge_index[0], (0, EPAD - E)).reshape(NW, NCH, CH)
    dst = jnp.pad(edge_index[1], (0, EPAD - E),
                  constant_values=NPAD - 1).reshape(NW, NCH, CH)
    xp = jnp.pad(x, ((0, NPAD - N), (0, 128 - x.shape[1])))
    eap = jnp.pad(edge_attr, ((0, EPAD - E), (0, 8 - edge_attr.shape[1])))
    ones_e = jnp.ones((EPAD, D), f32)
    zero_rows = jnp.zeros((NR, D), f32)
    rep = jnp.repeat(jnp.eye(D, dtype=jnp.bfloat16), D, axis=1)

    l0wt = jnp.pad(lin0_W.T, ((0, 128 - lin0_W.shape[1]), (0, 0)))
    l0b = jnp.broadcast_to(lin0_b, (8, D))
    w0t = jnp.pad(net0_W.T, ((0, 8 - net0_W.shape[1]), (0, 0)))
    b0 = jnp.broadcast_to(net0_b, (8, D))
    w2a = jnp.pad(jnp.concatenate([net2_W.T, net2_b[None, :]], axis=0),
                  ((0, 63), (0, 0))).astype(jnp.bfloat16)
    cb = jnp.broadcast_to(conv_b, (8, D))
    wir, wiz, win = (gru_Wih[i * D:(i + 1) * D].T for i in range(3))
    whr, whz, whn = (gru_Whh[i * D:(i + 1) * D].T for i in range(3))
    bir, biz, bin_ = (jnp.broadcast_to(gru_bih[i * D:(i + 1) * D], (8, D))
                      for i in range(3))
    bhr, bhz, bhn = (jnp.broadcast_to(gru_bhh[i * D:(i + 1) * D], (8, D))
                     for i in range(3))
    bih4 = jnp.pad(lstm_bih.reshape(4, D), ((0, 4), (0, 0)))
    bhh4 = jnp.pad(lstm_bhh.reshape(4, D), ((0, 4), (0, 0)))
    bpad = jnp.pad(batch, (0, NPAD - N), constant_values=2 * B)
    bcol = bpad.reshape(NPAD, 1)
    brow = bpad.reshape(SG, 1, ST)

    sc_gather, sc_scatter = _sc_kernels()
    cnt2 = sc_scatter(ones_e, dst, zero_rows).reshape(NCORES, NPAD, D)
    out, cinv = _nodes_prologue(xp, l0wt, l0b, cnt2)
    eh64 = _eh(eap, w0t, b0)
    eha = jnp.concatenate(
        [eh64, jnp.ones((EPAD, 1), jnp.bfloat16),
         jnp.zeros((EPAD, 63), jnp.bfloat16)], axis=1)
    wmat = _wmat(eha, w2a)

    for _ in range(6):
        xj = sc_gather(out, src)
        msg = _msg(wmat, xj, rep)
        aggr2 = sc_scatter(msg, dst, zero_rows).reshape(NCORES, NPAD, D)
        out = _gru(aggr2, cinv, out, root_W, cb,
                   wir, wiz, win, whr, whz, whn,
                   bir, biz, bin_, bhr, bhz, bhn)

    q, r = _set2set(out, bcol, brow, bih4, bhh4)
    return jnp.concatenate([q, r], axis=1)


# speedup vs baseline: 1.1786x; 1.1008x over previous
import functools

import jax
import jax.numpy as jnp
from jax import lax
from jax.experimental import pallas as pl
from jax.experimental.pallas import tpu as pltpu
from jax.experimental.pallas import tpu_sc as plsc

N = 5000
E = 10000
B = 128
D = 64

NPAD = 5120
EPAD = 10240
NCORES = 2
NSUB = 16
NW = NCORES * NSUB
EW = EPAD // NW
NR = NPAD // NSUB
CH = 64
NCH = EW // CH

ET = 512
EG = EPAD // ET
NT = 512
NG = NPAD // NT
ST = 640
SG = NPAD // ST


def _leaky(v):
    return jnp.where(v >= 0, v, 0.01 * v)


def _sc_gather_body(tbl_hbm, src_hbm, xj_hbm, idx_v, rows_v, sem):
    c = lax.axis_index("c")
    s = lax.axis_index("s")
    w = c * NSUB + s
    pltpu.sync_copy(src_hbm.at[w], idx_v)
    handles = [pltpu.async_copy(tbl_hbm.at[idx_v.at[j]],
                                rows_v.at[pl.ds(j * CH, CH)], sem)
               for j in range(NCH)]
    for h in handles:
        h.wait()
    pltpu.sync_copy(rows_v, xj_hbm.at[pl.ds(w * EW, EW)])


def _sc_scatter_body(msg_hbm, dst_hbm, zero_hbm, out_hbm,
                     idx_v, rows_v, tbl_sh, zsem, asem):
    c = lax.axis_index("c")
    s = lax.axis_index("s")
    w = c * NSUB + s
    zh = pltpu.async_copy(zero_hbm, tbl_sh.at[pl.ds(s * NR, NR)], zsem)
    pltpu.sync_copy(dst_hbm.at[w], idx_v)
    pltpu.sync_copy(msg_hbm.at[pl.ds(w * EW, EW)], rows_v)
    zh.wait()
    plsc.subcore_barrier()
    handles = [pltpu.async_copy(rows_v.at[pl.ds(j * CH, CH)],
                                tbl_sh.at[idx_v.at[j]], asem, add=True)
               for j in range(NCH)]
    for h in handles:
        h.wait()
    plsc.subcore_barrier()
    pltpu.sync_copy(tbl_sh.at[pl.ds(s * NR, NR)],
                    out_hbm.at[pl.ds(c * NPAD + s * NR, NR)])


@functools.lru_cache(maxsize=None)
def _sc_kernels():
    mesh = plsc.VectorSubcoreMesh(core_axis_name="c", subcore_axis_name="s",
                                  num_cores=NCORES, num_subcores=NSUB)
    params = pltpu.CompilerParams(use_tc_tiling_on_sc=False)
    gather = pl.kernel(
        _sc_gather_body,
        out_type=jax.ShapeDtypeStruct((EPAD, D), jnp.float32),
        mesh=mesh,
        compiler_params=params,
        scratch_types=[
            pltpu.VMEM((NCH, CH), jnp.int32),
            pltpu.VMEM((EW, D), jnp.float32),
            pltpu.SemaphoreType.DMA,
        ],
    )
    scatter = pl.kernel(
        _sc_scatter_body,
        out_type=jax.ShapeDtypeStruct((NCORES * NPAD, D), jnp.float32),
        mesh=mesh,
        compiler_params=params,
        scratch_types=[
            pltpu.VMEM((NCH, CH), jnp.int32),
            pltpu.VMEM((EW, D), jnp.float32),
            pltpu.VMEM_SHARED((NPAD, D), jnp.float32),
            pltpu.SemaphoreType.DMA,
            pltpu.SemaphoreType.DMA,
        ],
    )
    return gather, scatter


def _nodes_prologue_body(x_ref, w_ref, b_ref, cnt_ref, out_ref, cinv_ref):
    o = _leaky(jnp.dot(x_ref[...], w_ref[...],
                       preferred_element_type=jnp.float32) + b_ref[0:1])
    out_ref[...] = o
    c = cnt_ref[0, :, 0:1] + cnt_ref[1, :, 0:1]
    cinv_ref[...] = jnp.broadcast_to(1.0 / jnp.maximum(c, 1.0), (NT, 8))


def _eh_body(ea_ref, w0_ref, b0_ref, out_ref):
    eh = _leaky(jnp.dot(ea_ref[...], w0_ref[...],
                        preferred_element_type=jnp.float32) + b0_ref[0:1])
    out_ref[...] = eh.astype(jnp.bfloat16)


def _msg_body(eha_ref, xj_ref, rep_ref, w2a_ref, msg_ref, g_s, xb_s):
    g_s[...] = jnp.dot(eha_ref[...], w2a_ref[...],
                       preferred_element_type=jnp.float32)
    xb_s[...] = jnp.dot(xj_ref[...].astype(jnp.bfloat16), rep_ref[...],
                        preferred_element_type=jnp.float32)
    for rb in range(ET // 64):
        acc = jnp.zeros((64, 128), jnp.float32)
        for j in range(D // 2):
            acc += (xb_s[pl.ds(rb * 64, 64), pl.ds(j * 128, 128)]
                    * g_s[pl.ds(rb * 64, 64), pl.ds(j * 128, 128)])
        msg_ref[pl.ds(rb * 64, 64), :] = acc[:, :D] + acc[:, D:]


def _gru_body(aggr_ref, cinv_ref, st_ref, root_ref, cb_ref,
              wir_ref, wiz_ref, win_ref, whr_ref, whz_ref, whn_ref,
              bir_ref, biz_ref, bin_ref, bhr_ref, bhz_ref, bhn_ref,
              new_ref):
    a = (aggr_ref[0] + aggr_ref[1]) * cinv_ref[:, 0:1]
    o = st_ref[...]
    m = _leaky(a + jnp.dot(o, root_ref[...],
                           preferred_element_type=jnp.float32) + cb_ref[0:1])
    gr = (jnp.dot(m, wir_ref[...], preferred_element_type=jnp.float32)
          + bir_ref[0:1]
          + jnp.dot(o, whr_ref[...], preferred_element_type=jnp.float32)
          + bhr_ref[0:1])
    gz = (jnp.dot(m, wiz_ref[...], preferred_element_type=jnp.float32)
          + biz_ref[0:1]
          + jnp.dot(o, whz_ref[...], preferred_element_type=jnp.float32)
          + bhz_ref[0:1])
    r = jax.nn.sigmoid(gr)
    z = jax.nn.sigmoid(gz)
    gn = (jnp.dot(m, win_ref[...], preferred_element_type=jnp.float32)
          + bin_ref[0:1]
          + r * (jnp.dot(o, whn_ref[...], preferred_element_type=jnp.float32)
                 + bhn_ref[0:1]))
    n = jnp.tanh(gn)
    new_ref[...] = (1.0 - z) * n + z * o


def _set2set_body(st_ref, bcol_ref, brow_ref, bih_ref, bhh_ref,
                  q_ref, r_ref, emax_s, den_s, r_s):
    p = pl.program_id(0)
    i = pl.program_id(1)
    gb = bih_ref[...] + bhh_ref[...]
    cl = jax.nn.sigmoid(gb[0:1]) * jnp.tanh(gb[2:3])
    qrow = jax.nn.sigmoid(gb[3:4]) * jnp.tanh(cl)

    o = st_ref[...]
    bt = bcol_ref[...]
    oh = bt == lax.broadcasted_iota(jnp.int32, (ST, B), 1)
    e = jnp.sum(o * qrow, axis=1, keepdims=True)

    @pl.when(p == 0)
    def _pass_max():
        @pl.when(i == 0)
        def _():
            emax_s[...] = jnp.full((8, B), -1e30, jnp.float32)
        part = jnp.max(jnp.where(oh, e, -1e30), axis=0, keepdims=True)
        emax_s[...] = jnp.maximum(emax_s[...], jnp.broadcast_to(part, (8, B)))

    @pl.when(p == 1)
    def _pass_read():
        @pl.when(i == 0)
        def _():
            den_s[...] = jnp.zeros((B, 8), jnp.float32)
            r_s[...] = jnp.zeros((B, D), jnp.float32)
        em = emax_s[0:1]
        eb = jnp.sum(jnp.where(oh, jnp.broadcast_to(em, (ST, B)), 0.0),
                     axis=1, keepdims=True)
        a = jnp.where(bt < B, jnp.exp(e - eb), 0.0)
        btr = brow_ref[0]
        oht = (lax.broadcasted_iota(jnp.int32, (B, ST), 0) == btr
               ).astype(jnp.float32)
        den_s[:, 0:1] += jnp.dot(oht, a, preferred_element_type=jnp.float32)
        r_s[...] += jnp.dot(oht, a * o, preferred_element_type=jnp.float32)

        @pl.when(i == SG - 1)
        def _():
            d = jnp.maximum(den_s[:, 0:1], 1e-16)
            r_ref[...] = r_s[...] / d
            q_ref[...] = jnp.broadcast_to(qrow, (B, D))


def _full(shape):
    return pl.BlockSpec(shape, lambda *_: tuple(0 for _ in shape))


_nodes_prologue = pl.pallas_call(
    _nodes_prologue_body,
    grid=(NG,),
    in_specs=[
        pl.BlockSpec((NT, 128), lambda i: (i, 0)),
        _full((128, D)),
        _full((8, D)),
        pl.BlockSpec((2, NT, D), lambda i: (0, i, 0)),
    ],
    out_specs=[
        pl.BlockSpec((NT, D), lambda i: (i, 0)),
        pl.BlockSpec((NT, 8), lambda i: (i, 0)),
    ],
    out_shape=[
        jax.ShapeDtypeStruct((NPAD, D), jnp.float32),
        jax.ShapeDtypeStruct((NPAD, 8), jnp.float32),
    ],
)

_eh = pl.pallas_call(
    _eh_body,
    grid=(EG,),
    in_specs=[
        pl.BlockSpec((ET, 8), lambda i: (i, 0)),
        _full((8, D)),
        _full((8, D)),
    ],
    out_specs=pl.BlockSpec((ET, D), lambda i: (i, 0)),
    out_shape=jax.ShapeDtypeStruct((EPAD, D), jnp.bfloat16),
)

_msg = pl.pallas_call(
    _msg_body,
    grid=(EG,),
    in_specs=[
        pl.BlockSpec((ET, 128), lambda i: (i, 0)),
        pl.BlockSpec((ET, D), lambda i: (i, 0)),
        _full((D, D * D)),
        _full((128, D * D)),
    ],
    out_specs=pl.BlockSpec((ET, D), lambda i: (i, 0)),
    out_shape=jax.ShapeDtypeStruct((EPAD, D), jnp.float32),
    scratch_shapes=[pltpu.VMEM((ET, D * D), jnp.float32),
                    pltpu.VMEM((ET, D * D), jnp.float32)],
)

_gru = pl.pallas_call(
    _gru_body,
    grid=(NG,),
    in_specs=[
        pl.BlockSpec((2, NT, D), lambda i: (0, i, 0)),
        pl.BlockSpec((NT, 8), lambda i: (i, 0)),
        pl.BlockSpec((NT, D), lambda i: (i, 0)),
        _full((D, D)), _full((8, D)),
        _full((D, D)), _full((D, D)), _full((D, D)),
        _full((D, D)), _full((D, D)), _full((D, D)),
        _full((8, D)), _full((8, D)), _full((8, D)),
        _full((8, D)), _full((8, D)), _full((8, D)),
    ],
    out_specs=pl.BlockSpec((NT, D), lambda i: (i, 0)),
    out_shape=jax.ShapeDtypeStruct((NPAD, D), jnp.float32),
)

_set2set = pl.pallas_call(
    _set2set_body,
    grid=(2, SG),
    in_specs=[
        pl.BlockSpec((ST, D), lambda p, i: (i, 0)),
        pl.BlockSpec((ST, 1), lambda p, i: (i, 0)),
        pl.BlockSpec((1, 1, ST), lambda p, i: (i, 0, 0)),
        _full((8, D)),
        _full((8, D)),
    ],
    out_specs=[
        pl.BlockSpec((B, D), lambda p, i: (0, 0)),
        pl.BlockSpec((B, D), lambda p, i: (0, 0)),
    ],
    out_shape=[
        jax.ShapeDtypeStruct((B, D), jnp.float32),
        jax.ShapeDtypeStruct((B, D), jnp.float32),
    ],
    scratch_shapes=[
        pltpu.VMEM((8, B), jnp.float32),
        pltpu.VMEM((B, 8), jnp.float32),
        pltpu.VMEM((B, D), jnp.float32),
    ],
)


def kernel(x, edge_index, edge_attr, batch, lin0_W, lin0_b, net0_W, net0_b,
           net2_W, net2_b, root_W, conv_b, gru_Wih, gru_Whh, gru_bih, gru_bhh,
           lstm_Wih, lstm_Whh, lstm_bih, lstm_bhh):
    f32 = jnp.float32

    src = jnp.pad(edge_index[0], (0, EPAD - E)).reshape(NW, NCH, CH)
    dst = jnp.pad(edge_index[1], (0, EPAD - E),
                  constant_values=NPAD - 1).reshape(NW, NCH, CH)
    xp = jnp.pad(x, ((0, NPAD - N), (0, 128 - x.shape[1])))
    eap = jnp.pad(edge_attr, ((0, EPAD - E), (0, 8 - edge_attr.shape[1])))
    ones_e = jnp.ones((EPAD, D), f32)
    zero_rows = jnp.zeros((NR, D), f32)
    rep = jnp.repeat(jnp.eye(D, dtype=jnp.bfloat16), D, axis=1)

    l0wt = jnp.pad(lin0_W.T, ((0, 128 - lin0_W.shape[1]), (0, 0)))
    l0b = jnp.broadcast_to(lin0_b, (8, D))
    w0t = jnp.pad(net0_W.T, ((0, 8 - net0_W.shape[1]), (0, 0)))
    b0 = jnp.broadcast_to(net0_b, (8, D))
    w2a = jnp.pad(jnp.concatenate([net2_W.T, net2_b[None, :]], axis=0),
                  ((0, 63), (0, 0))).astype(jnp.bfloat16)
    cb = jnp.broadcast_to(conv_b, (8, D))
    wir, wiz, win = (gru_Wih[i * D:(i + 1) * D].T for i in range(3))
    whr, whz, whn = (gru_Whh[i * D:(i + 1) * D].T for i in range(3))
    bir, biz, bin_ = (jnp.broadcast_to(gru_bih[i * D:(i + 1) * D], (8, D))
                      for i in range(3))
    bhr, bhz, bhn = (jnp.broadcast_to(gru_bhh[i * D:(i + 1) * D], (8, D))
                     for i in range(3))
    bih4 = jnp.pad(lstm_bih.reshape(4, D), ((0, 4), (0, 0)))
    bhh4 = jnp.pad(lstm_bhh.reshape(4, D), ((0, 4), (0, 0)))
    bpad = jnp.pad(batch, (0, NPAD - N), constant_values=2 * B)
    bcol = bpad.reshape(NPAD, 1)
    brow = bpad.reshape(SG, 1, ST)

    sc_gather, sc_scatter = _sc_kernels()
    cnt2 = sc_scatter(ones_e, dst, zero_rows).reshape(NCORES, NPAD, D)
    out, cinv = _nodes_prologue(xp, l0wt, l0b, cnt2)
    eh64 = _eh(eap, w0t, b0)
    eha = jnp.concatenate(
        [eh64, jnp.ones((EPAD, 1), jnp.bfloat16),
         jnp.zeros((EPAD, 63), jnp.bfloat16)], axis=1)
    for _ in range(6):
        xj = sc_gather(out, src)
        msg = _msg(eha, xj, rep, w2a)
        aggr2 = sc_scatter(msg, dst, zero_rows).reshape(NCORES, NPAD, D)
        out = _gru(aggr2, cinv, out, root_W, cb,
                   wir, wiz, win, whr, whz, whn,
                   bir, biz, bin_, bhr, bhz, bhn)

    q, r = _set2set(out, bcol, brow, bih4, bhh4)
    return jnp.concatenate([q, r], axis=1)


# speedup vs baseline: 1.2109x; 1.0274x over previous
import functools

import jax
import jax.numpy as jnp
from jax import lax
from jax.experimental import pallas as pl
from jax.experimental.pallas import tpu as pltpu
from jax.experimental.pallas import tpu_sc as plsc

N = 5000
E = 10000
B = 128
D = 64

NPAD = 5120
EPAD = 10240
NCORES = 2
NSUB = 16
NW = NCORES * NSUB
EW = EPAD // NW
NR = NPAD // NSUB
CH = 64
NCH = EW // CH

ET = 1024
EG = EPAD // ET
NT = 512
NG = NPAD // NT
ST = 640
SG = NPAD // ST


def _leaky(v):
    return jnp.where(v >= 0, v, 0.01 * v)


def _sc_gather_body(tbl_hbm, src_hbm, xj_hbm, idx_v, rows_v, sem):
    c = lax.axis_index("c")
    s = lax.axis_index("s")
    w = c * NSUB + s
    pltpu.sync_copy(src_hbm.at[w], idx_v)
    handles = [pltpu.async_copy(tbl_hbm.at[idx_v.at[j]],
                                rows_v.at[pl.ds(j * CH, CH)], sem)
               for j in range(NCH)]
    for h in handles:
        h.wait()
    pltpu.sync_copy(rows_v, xj_hbm.at[pl.ds(w * EW, EW)])


def _sc_scatter_body(msg_hbm, dst_hbm, zero_hbm, out_hbm,
                     idx_v, rows_v, tbl_sh, zsem, asem):
    c = lax.axis_index("c")
    s = lax.axis_index("s")
    w = c * NSUB + s
    zh = pltpu.async_copy(zero_hbm, tbl_sh.at[pl.ds(s * NR, NR)], zsem)
    pltpu.sync_copy(dst_hbm.at[w], idx_v)
    pltpu.sync_copy(msg_hbm.at[pl.ds(w * EW, EW)], rows_v)
    zh.wait()
    plsc.subcore_barrier()
    handles = [pltpu.async_copy(rows_v.at[pl.ds(j * CH, CH)],
                                tbl_sh.at[idx_v.at[j]], asem, add=True)
               for j in range(NCH)]
    for h in handles:
        h.wait()
    plsc.subcore_barrier()
    pltpu.sync_copy(tbl_sh.at[pl.ds(s * NR, NR)],
                    out_hbm.at[pl.ds(c * NPAD + s * NR, NR)])


@functools.lru_cache(maxsize=None)
def _sc_kernels():
    mesh = plsc.VectorSubcoreMesh(core_axis_name="c", subcore_axis_name="s",
                                  num_cores=NCORES, num_subcores=NSUB)
    params = pltpu.CompilerParams(use_tc_tiling_on_sc=False)
    gather = pl.kernel(
        _sc_gather_body,
        out_type=jax.ShapeDtypeStruct((EPAD, D), jnp.float32),
        mesh=mesh,
        compiler_params=params,
        scratch_types=[
            pltpu.VMEM((NCH, CH), jnp.int32),
            pltpu.VMEM((EW, D), jnp.float32),
            pltpu.SemaphoreType.DMA,
        ],
    )
    scatter = pl.kernel(
        _sc_scatter_body,
        out_type=jax.ShapeDtypeStruct((NCORES * NPAD, D), jnp.float32),
        mesh=mesh,
        compiler_params=params,
        scratch_types=[
            pltpu.VMEM((NCH, CH), jnp.int32),
            pltpu.VMEM((EW, D), jnp.float32),
            pltpu.VMEM_SHARED((NPAD, D), jnp.float32),
            pltpu.SemaphoreType.DMA,
            pltpu.SemaphoreType.DMA,
        ],
    )
    return gather, scatter


def _nodes_prologue_body(x_ref, w_ref, b_ref, cnt_ref, out_ref, cinv_ref):
    o = _leaky(jnp.dot(x_ref[...], w_ref[...],
                       preferred_element_type=jnp.float32) + b_ref[0:1])
    out_ref[...] = o
    c = cnt_ref[0, :, 0:1] + cnt_ref[1, :, 0:1]
    cinv_ref[...] = jnp.broadcast_to(1.0 / jnp.maximum(c, 1.0), (NT, 8))


def _eh_body(ea_ref, w0_ref, b0_ref, out_ref):
    eh = _leaky(jnp.dot(ea_ref[...], w0_ref[...],
                        preferred_element_type=jnp.float32) + b0_ref[0:1])
    out_ref[...] = eh.astype(jnp.bfloat16)


def _msg_body(eha_ref, xj_ref, rep_ref, w2a_ref, msg_ref, g_s, xb_s):
    g_s[...] = jnp.dot(eha_ref[...], w2a_ref[...],
                       preferred_element_type=jnp.float32)
    xb_s[...] = jnp.dot(xj_ref[...].astype(jnp.bfloat16), rep_ref[...],
                        preferred_element_type=jnp.float32)
    for rb in range(ET // 64):
        acc = jnp.zeros((64, 128), jnp.float32)
        for j in range(D // 2):
            acc += (xb_s[pl.ds(rb * 64, 64), pl.ds(j * 128, 128)]
                    * g_s[pl.ds(rb * 64, 64), pl.ds(j * 128, 128)])
        msg_ref[pl.ds(rb * 64, 64), :] = acc[:, :D] + acc[:, D:]


def _gru_body(aggr_ref, cinv_ref, st_ref, root_ref, cb_ref,
              wir_ref, wiz_ref, win_ref, whr_ref, whz_ref, whn_ref,
              bir_ref, biz_ref, bin_ref, bhr_ref, bhz_ref, bhn_ref,
              new_ref):
    a = (aggr_ref[0] + aggr_ref[1]) * cinv_ref[:, 0:1]
    o = st_ref[...]
    m = _leaky(a + jnp.dot(o, root_ref[...],
                           preferred_element_type=jnp.float32) + cb_ref[0:1])
    gr = (jnp.dot(m, wir_ref[...], preferred_element_type=jnp.float32)
          + bir_ref[0:1]
          + jnp.dot(o, whr_ref[...], preferred_element_type=jnp.float32)
          + bhr_ref[0:1])
    gz = (jnp.dot(m, wiz_ref[...], preferred_element_type=jnp.float32)
          + biz_ref[0:1]
          + jnp.dot(o, whz_ref[...], preferred_element_type=jnp.float32)
          + bhz_ref[0:1])
    r = jax.nn.sigmoid(gr)
    z = jax.nn.sigmoid(gz)
    gn = (jnp.dot(m, win_ref[...], preferred_element_type=jnp.float32)
          + bin_ref[0:1]
          + r * (jnp.dot(o, whn_ref[...], preferred_element_type=jnp.float32)
                 + bhn_ref[0:1]))
    n = jnp.tanh(gn)
    new_ref[...] = (1.0 - z) * n + z * o


def _set2set_body(st_ref, bcol_ref, brow_ref, bih_ref, bhh_ref,
                  q_ref, r_ref, emax_s, den_s, r_s):
    p = pl.program_id(0)
    i = pl.program_id(1)
    gb = bih_ref[...] + bhh_ref[...]
    cl = jax.nn.sigmoid(gb[0:1]) * jnp.tanh(gb[2:3])
    qrow = jax.nn.sigmoid(gb[3:4]) * jnp.tanh(cl)

    o = st_ref[...]
    bt = bcol_ref[...]
    oh = bt == lax.broadcasted_iota(jnp.int32, (ST, B), 1)
    e = jnp.sum(o * qrow, axis=1, keepdims=True)

    @pl.when(p == 0)
    def _pass_max():
        @pl.when(i == 0)
        def _():
            emax_s[...] = jnp.full((8, B), -1e30, jnp.float32)
        part = jnp.max(jnp.where(oh, e, -1e30), axis=0, keepdims=True)
        emax_s[...] = jnp.maximum(emax_s[...], jnp.broadcast_to(part, (8, B)))

    @pl.when(p == 1)
    def _pass_read():
        @pl.when(i == 0)
        def _():
            den_s[...] = jnp.zeros((B, 8), jnp.float32)
            r_s[...] = jnp.zeros((B, D), jnp.float32)
        em = emax_s[0:1]
        eb = jnp.sum(jnp.where(oh, jnp.broadcast_to(em, (ST, B)), 0.0),
                     axis=1, keepdims=True)
        a = jnp.where(bt < B, jnp.exp(e - eb), 0.0)
        btr = brow_ref[0]
        oht = (lax.broadcasted_iota(jnp.int32, (B, ST), 0) == btr
               ).astype(jnp.float32)
        den_s[:, 0:1] += jnp.dot(oht, a, preferred_element_type=jnp.float32)
        r_s[...] += jnp.dot(oht, a * o, preferred_element_type=jnp.float32)

        @pl.when(i == SG - 1)
        def _():
            d = jnp.maximum(den_s[:, 0:1], 1e-16)
            r_ref[...] = r_s[...] / d
            q_ref[...] = jnp.broadcast_to(qrow, (B, D))


def _full(shape):
    return pl.BlockSpec(shape, lambda *_: tuple(0 for _ in shape))


_nodes_prologue = pl.pallas_call(
    _nodes_prologue_body,
    grid=(NG,),
    in_specs=[
        pl.BlockSpec((NT, 128), lambda i: (i, 0)),
        _full((128, D)),
        _full((8, D)),
        pl.BlockSpec((2, NT, D), lambda i: (0, i, 0)),
    ],
    out_specs=[
        pl.BlockSpec((NT, D), lambda i: (i, 0)),
        pl.BlockSpec((NT, 8), lambda i: (i, 0)),
    ],
    out_shape=[
        jax.ShapeDtypeStruct((NPAD, D), jnp.float32),
        jax.ShapeDtypeStruct((NPAD, 8), jnp.float32),
    ],
)

_eh = pl.pallas_call(
    _eh_body,
    grid=(EG,),
    in_specs=[
        pl.BlockSpec((ET, 8), lambda i: (i, 0)),
        _full((8, D)),
        _full((8, D)),
    ],
    out_specs=pl.BlockSpec((ET, D), lambda i: (i, 0)),
    out_shape=jax.ShapeDtypeStruct((EPAD, D), jnp.bfloat16),
)

_msg = pl.pallas_call(
    _msg_body,
    grid=(EG,),
    in_specs=[
        pl.BlockSpec((ET, 128), lambda i: (i, 0)),
        pl.BlockSpec((ET, D), lambda i: (i, 0)),
        _full((D, D * D)),
        _full((128, D * D)),
    ],
    out_specs=pl.BlockSpec((ET, D), lambda i: (i, 0)),
    out_shape=jax.ShapeDtypeStruct((EPAD, D), jnp.float32),
    scratch_shapes=[pltpu.VMEM((ET, D * D), jnp.float32),
                    pltpu.VMEM((ET, D * D), jnp.float32)],
)

_gru = pl.pallas_call(
    _gru_body,
    grid=(NG,),
    in_specs=[
        pl.BlockSpec((2, NT, D), lambda i: (0, i, 0)),
        pl.BlockSpec((NT, 8), lambda i: (i, 0)),
        pl.BlockSpec((NT, D), lambda i: (i, 0)),
        _full((D, D)), _full((8, D)),
        _full((D, D)), _full((D, D)), _full((D, D)),
        _full((D, D)), _full((D, D)), _full((D, D)),
        _full((8, D)), _full((8, D)), _full((8, D)),
        _full((8, D)), _full((8, D)), _full((8, D)),
    ],
    out_specs=pl.BlockSpec((NT, D), lambda i: (i, 0)),
    out_shape=jax.ShapeDtypeStruct((NPAD, D), jnp.float32),
)

_set2set = pl.pallas_call(
    _set2set_body,
    grid=(2, SG),
    in_specs=[
        pl.BlockSpec((ST, D), lambda p, i: (i, 0)),
        pl.BlockSpec((ST, 1), lambda p, i: (i, 0)),
        pl.BlockSpec((1, 1, ST), lambda p, i: (i, 0, 0)),
        _full((8, D)),
        _full((8, D)),
    ],
    out_specs=[
        pl.BlockSpec((B, D), lambda p, i: (0, 0)),
        pl.BlockSpec((B, D), lambda p, i: (0, 0)),
    ],
    out_shape=[
        jax.ShapeDtypeStruct((B, D), jnp.float32),
        jax.ShapeDtypeStruct((B, D), jnp.float32),
    ],
    scratch_shapes=[
        pltpu.VMEM((8, B), jnp.float32),
        pltpu.VMEM((B, 8), jnp.float32),
        pltpu.VMEM((B, D), jnp.float32),
    ],
)


def kernel(x, edge_index, edge_attr, batch, lin0_W, lin0_b, net0_W, net0_b,
           net2_W, net2_b, root_W, conv_b, gru_Wih, gru_Whh, gru_bih, gru_bhh,
           lstm_Wih, lstm_Whh, lstm_bih, lstm_bhh):
    f32 = jnp.float32

    src = jnp.pad(edge_index[0], (0, EPAD - E)).reshape(NW, NCH, CH)
    dst = jnp.pad(edge_index[1], (0, EPAD - E),
                  constant_values=NPAD - 1).reshape(NW, NCH, CH)
    xp = jnp.pad(x, ((0, NPAD - N), (0, 128 - x.shape[1])))
    eap = jnp.pad(edge_attr, ((0, EPAD - E), (0, 8 - edge_attr.shape[1])))
    ones_e = jnp.ones((EPAD, D), f32)
    zero_rows = jnp.zeros((NR, D), f32)
    rep = jnp.repeat(jnp.eye(D, dtype=jnp.bfloat16), D, axis=1)

    l0wt = jnp.pad(lin0_W.T, ((0, 128 - lin0_W.shape[1]), (0, 0)))
    l0b = jnp.broadcast_to(lin0_b, (8, D))
    w0t = jnp.pad(net0_W.T, ((0, 8 - net0_W.shape[1]), (0, 0)))
    b0 = jnp.broadcast_to(net0_b, (8, D))
    w2a = jnp.pad(jnp.concatenate([net2_W.T, net2_b[None, :]], axis=0),
                  ((0, 63), (0, 0))).astype(jnp.bfloat16)
    cb = jnp.broadcast_to(conv_b, (8, D))
    wir, wiz, win = (gru_Wih[i * D:(i + 1) * D].T for i in range(3))
    whr, whz, whn = (gru_Whh[i * D:(i + 1) * D].T for i in range(3))
    bir, biz, bin_ = (jnp.broadcast_to(gru_bih[i * D:(i + 1) * D], (8, D))
                      for i in range(3))
    bhr, bhz, bhn = (jnp.broadcast_to(gru_bhh[i * D:(i + 1) * D], (8, D))
                     for i in range(3))
    bih4 = jnp.pad(lstm_bih.reshape(4, D), ((0, 4), (0, 0)))
    bhh4 = jnp.pad(lstm_bhh.reshape(4, D), ((0, 4), (0, 0)))
    bpad = jnp.pad(batch, (0, NPAD - N), constant_values=2 * B)
    bcol = bpad.reshape(NPAD, 1)
    brow = bpad.reshape(SG, 1, ST)

    sc_gather, sc_scatter = _sc_kernels()
    cnt2 = sc_scatter(ones_e, dst, zero_rows).reshape(NCORES, NPAD, D)
    out, cinv = _nodes_prologue(xp, l0wt, l0b, cnt2)
    eh64 = _eh(eap, w0t, b0)
    eha = jnp.concatenate(
        [eh64, jnp.ones((EPAD, 1), jnp.bfloat16),
         jnp.zeros((EPAD, 63), jnp.bfloat16)], axis=1)
    for _ in range(6):
        xj = sc_gather(out, src)
        msg = _msg(eha, xj, rep, w2a)
        aggr2 = sc_scatter(msg, dst, zero_rows).reshape(NCORES, NPAD, D)
        out = _gru(aggr2, cinv, out, root_W, cb,
                   wir, wiz, win, whr, whz, whn,
                   bir, biz, bin_, bhr, bhz, bhn)

    q, r = _set2set(out, bcol, brow, bih4, bhh4)
    return jnp.concatenate([q, r], axis=1)
